# baseline (speedup 1.0000x reference)
import functools
import jax
import jax.numpy as jnp
from jax import lax
from jax.experimental import pallas as pl
from jax.experimental.pallas import tpu as pltpu
from jax.experimental.pallas import tpu_sc as plsc

DIM = 64
BATCH = 16384
SEQ = 50
VOCAB_PAIRS = 500000
NW = 32
BB = 128
NCHUNK = SEQ * (BATCH // BB)
PER_W = NCHUNK // NW
TCB = BATCH // BB


@functools.cache
def _build():
    mesh = plsc.VectorSubcoreMesh(core_axis_name="c", subcore_axis_name="s")
    return pl.kernel(
        _emb_lookup,
        mesh=mesh,
        out_type=jax.ShapeDtypeStruct((SEQ * 8 * TCB, 8, BB), jnp.float32),
        scratch_types=[
            pltpu.VMEM((1, BB), jnp.int32),
            pltpu.VMEM((1, BB), jnp.int32),
            pltpu.VMEM((1, BB), jnp.int32),
            pltpu.VMEM((BB, BB), jnp.float32),
            pltpu.VMEM((8, 8, BB), jnp.float32),
            pltpu.SemaphoreType.DMA,
        ],
        compiler_params=pltpu.CompilerParams(
            use_tc_tiling_on_sc=False, needs_layout_passes=False
        ),
    )


def _emb_lookup(xt_hbm, tw_hbm, out_hbm, idxraw, pairs, halves, rows, ostage, sem):
    cid = lax.axis_index("c")
    sid = lax.axis_index("s")
    wid = sid * 2 + cid
    c0 = wid * PER_W
    iota = lax.iota(jnp.int32, 16)

    def chunk_body(g, carry):
        c = c0 + g
        s = c // TCB
        tc = c % TCB

        pltpu.sync_copy(xt_hbm.at[pl.ds(s, 1), pl.ds(tc * BB, BB)], idxraw)
        for k in range(BB // 16):
            v = idxraw[0, pl.ds(16 * k, 16)]
            pairs[0, pl.ds(16 * k, 16)] = lax.shift_right_logical(v, 1)
            halves[0, pl.ds(16 * k, 16)] = (v & 1) * DIM
        pltpu.async_copy(tw_hbm.at[pairs.at[0]], rows, sem).wait()

        def bbq_body(q, c2):
            bb16 = 16 * q + iota
            hv16 = halves[0, pl.ds(16 * q, 16)]
            for tr in range(8):
                for dd in range(8):
                    val = plsc.load_gather(rows, [bb16, hv16 + (8 * tr + dd)])
                    ostage[tr, dd, pl.ds(16 * q, 16)] = val * 8.0
            return c2

        lax.fori_loop(0, 8, bbq_body, 0)

        for tr in range(8):
            pltpu.sync_copy(
                ostage.at[pl.ds(tr, 1)],
                out_hbm.at[pl.ds(s * (8 * TCB) + tr * TCB + tc, 1)],
            )
        return carry

    lax.fori_loop(0, PER_W, chunk_body, 0)


def kernel(x, table):
    xt = x.T.astype(jnp.int32)
    tw = table.reshape(VOCAB_PAIRS, 2 * DIM)
    out3 = _build()(xt, tw)
    out5 = out3.reshape(SEQ, 8, TCB, 8, BB)
    return out5.transpose(2, 4, 0, 1, 3).reshape(BATCH, SEQ, DIM)


# speedup vs baseline: 1.1241x; 1.1241x over previous
import functools
import jax
import jax.numpy as jnp
from jax import lax
from jax.experimental import pallas as pl
from jax.experimental.pallas import tpu as pltpu
from jax.experimental.pallas import tpu_sc as plsc

DIM = 64
BATCH = 16384
SEQ = 50
PITCH = 72
NW = 32
BB = 128
NCHUNK = SEQ * (BATCH // BB)
PER_W = NCHUNK // NW
TCB = BATCH // BB


@functools.cache
def _build():
    mesh = plsc.VectorSubcoreMesh(core_axis_name="c", subcore_axis_name="s")
    return pl.kernel(
        _emb_lookup,
        mesh=mesh,
        out_type=jax.ShapeDtypeStruct((SEQ * 8 * TCB, 8, BB), jnp.float32),
        scratch_types=[
            pltpu.VMEM((1, BB), jnp.int32),
            pltpu.VMEM((BB, PITCH), jnp.float32),
            pltpu.VMEM((8, 8, BB), jnp.float32),
            pltpu.SemaphoreType.DMA,
        ],
        compiler_params=pltpu.CompilerParams(
            use_tc_tiling_on_sc=False, needs_layout_passes=False
        ),
    )


def _emb_lookup(xt_hbm, table_hbm, out_hbm, idx_v, rows, ostage, sem):
    cid = lax.axis_index("c")
    sid = lax.axis_index("s")
    wid = sid * 2 + cid
    c0 = wid * PER_W
    iota = lax.iota(jnp.int32, 16)

    def chunk_body(g, carry):
        c = c0 + g
        s = c // TCB
        tc = c % TCB

        pltpu.sync_copy(xt_hbm.at[pl.ds(s, 1), pl.ds(tc * BB, BB)], idx_v)
        pltpu.async_copy(table_hbm.at[idx_v.at[0]], rows, sem).wait()

        def bbq_body(q, c2):
            bb16 = 16 * q + iota
            for tr in range(8):
                for dd in range(8):
                    d16 = jnp.full((16,), 8 * tr + dd, jnp.int32)
                    val = plsc.load_gather(rows, [bb16, d16])
                    ostage[tr, dd, pl.ds(16 * q, 16)] = val * 8.0
            return c2

        lax.fori_loop(0, 8, bbq_body, 0)

        for tr in range(8):
            pltpu.sync_copy(
                ostage.at[pl.ds(tr, 1)],
                out_hbm.at[pl.ds(s * (8 * TCB) + tr * TCB + tc, 1)],
            )
        return carry

    lax.fori_loop(0, PER_W, chunk_body, 0)


def kernel(x, table):
    xt = x.T.astype(jnp.int32)
    tw = jnp.pad(table, ((0, 0), (0, PITCH - DIM)))
    out3 = _build()(xt, tw)
    out5 = out3.reshape(SEQ, 8, TCB, 8, BB)
    return out5.transpose(2, 4, 0, 1, 3).reshape(BATCH, SEQ, DIM)


# speedup vs baseline: 1.3023x; 1.1585x over previous
import functools
import jax
import jax.numpy as jnp
from jax import lax
from jax.experimental import pallas as pl
from jax.experimental.pallas import tpu as pltpu
from jax.experimental.pallas import tpu_sc as plsc

DIM = 64
BATCH = 16384
SEQ = 50
SKEW = 65
NW = 32
BB = 128
NCHUNK = SEQ * (BATCH // BB)
PER_W = NCHUNK // NW
TCB = BATCH // BB


@functools.cache
def _build():
    mesh = plsc.VectorSubcoreMesh(core_axis_name="c", subcore_axis_name="s")
    return pl.kernel(
        _emb_lookup,
        mesh=mesh,
        out_type=jax.ShapeDtypeStruct((SEQ * 8 * TCB, 8, BB), jnp.float32),
        scratch_types=[
            pltpu.VMEM((1, BB), jnp.int32),
            pltpu.VMEM((1, BB), jnp.int32),
            pltpu.VMEM((BB, DIM), jnp.float32),
            pltpu.VMEM((BB * SKEW,), jnp.float32),
            pltpu.VMEM((8, 8, BB), jnp.float32),
            pltpu.SemaphoreType.DMA,
        ],
        compiler_params=pltpu.CompilerParams(
            use_tc_tiling_on_sc=False, needs_layout_passes=False
        ),
    )


def _emb_lookup(xt_hbm, t2_hbm, out_hbm, idx_v, dbl_v, rows, skew, ostage, sem):
    cid = lax.axis_index("c")
    sid = lax.axis_index("s")
    wid = sid * 2 + cid
    c0 = wid * PER_W
    iota = lax.iota(jnp.int32, 16)

    def chunk_body(g, carry):
        c = c0 + g
        s = c // TCB
        tc = c % TCB

        pltpu.sync_copy(xt_hbm.at[pl.ds(s, 1), pl.ds(tc * BB, BB)], idx_v)
        for k in range(BB // 16):
            dbl_v[0, pl.ds(16 * k, 16)] = idx_v[0, pl.ds(16 * k, 16)] * 2
        pltpu.async_copy(t2_hbm.at[dbl_v.at[0]], rows, sem).wait()

        def row_body(bb, c2):
            base = bb * SKEW
            for j in range(DIM // 16):
                val = rows[bb, pl.ds(16 * j, 16)] * 8.0
                plsc.store_scatter(skew, [base + 16 * j + iota], val)
            return c2

        lax.fori_loop(0, BB, row_body, 0, unroll=2)

        def bbq_body(q, c2):
            bbp = (16 * q + iota) * SKEW
            for tr in range(8):
                for dd in range(8):
                    val = plsc.load_gather(skew, [bbp + (8 * tr + dd)])
                    ostage[tr, dd, pl.ds(16 * q, 16)] = val
            return c2

        lax.fori_loop(0, 8, bbq_body, 0)

        for tr in range(8):
            pltpu.sync_copy(
                ostage.at[pl.ds(tr, 1)],
                out_hbm.at[pl.ds(s * (8 * TCB) + tr * TCB + tc, 1)],
            )
        return carry

    lax.fori_loop(0, PER_W, chunk_body, 0)


def kernel(x, table):
    xt = x.T.astype(jnp.int32)
    twide = jnp.pad(table, ((0, 0), (0, DIM)))
    t2 = twide.reshape(2 * 1000000, DIM)
    out3 = _build()(xt, t2)
    out5 = out3.reshape(SEQ, 8, TCB, 8, BB)
    return out5.transpose(2, 4, 0, 1, 3).reshape(BATCH, SEQ, DIM)


# speedup vs baseline: 2.7911x; 2.1432x over previous
import functools
import jax
import jax.numpy as jnp
from jax import lax
from jax.experimental import pallas as pl
from jax.experimental.pallas import tpu as pltpu
from jax.experimental.pallas import tpu_sc as plsc

DIM = 64
BATCH = 16384
SEQ = 50
SKEW = 65
NW = 32
BB = 128
GPC = 4
CB = BB * GPC
NCHUNK = SEQ * (BATCH // CB)
PER_W = NCHUNK // NW
TCB = BATCH // BB
TQ = BATCH // CB


@functools.cache
def _build():
    mesh = plsc.VectorSubcoreMesh(core_axis_name="c", subcore_axis_name="s")
    return pl.kernel(
        _emb_lookup,
        mesh=mesh,
        out_type=jax.ShapeDtypeStruct((SEQ * 8 * TCB, 8, BB), jnp.float32),
        scratch_types=[
            pltpu.VMEM((1, CB), jnp.int32),
            pltpu.VMEM((GPC, BB), jnp.int32),
            pltpu.VMEM((CB, DIM), jnp.float32),
            pltpu.VMEM((CB * SKEW,), jnp.float32),
            pltpu.VMEM((4 * 8, 8, BB), jnp.float32),
            pltpu.SemaphoreType.DMA,
        ],
        compiler_params=pltpu.CompilerParams(
            use_tc_tiling_on_sc=False, needs_layout_passes=False
        ),
    )


def _emb_lookup(xt_hbm, t2_hbm, out_hbm, idx_v, dbl_v, rows, skew, ostage, sem):
    cid = lax.axis_index("c")
    sid = lax.axis_index("s")
    wid = sid * 2 + cid
    c0 = wid * PER_W
    iota = lax.iota(jnp.int32, 16)
    iota65 = iota * SKEW

    def chunk_body(g, carry):
        c = c0 + g
        s = c // TQ
        tq = c % TQ

        pltpu.sync_copy(xt_hbm.at[pl.ds(s, 1), pl.ds(tq * CB, CB)], idx_v)
        for j in range(GPC):
            for k in range(BB // 16):
                dbl_v[j, pl.ds(16 * k, 16)] = (
                    idx_v[0, pl.ds(j * BB + 16 * k, 16)] * 2
                )
        copies = [
            pltpu.async_copy(
                t2_hbm.at[dbl_v.at[j]], rows.at[pl.ds(j * BB, BB)], sem
            )
            for j in range(GPC)
        ]
        for cp in copies:
            cp.wait()

        @plsc.parallel_loop(0, CB, unroll=4)
        def _pass1(bb):
            base = bb * SKEW
            for j in range(DIM // 16):
                val = rows[bb, pl.ds(16 * j, 16)] * 8.0
                plsc.store_scatter(skew, [base + 16 * j + iota], val)

        @plsc.parallel_loop(0, 8 * GPC * 8, unroll=2)
        def _pass2(o):
            row = o // 8
            qq = o % 8
            tr = row // GPC
            j4 = row % GPC
            base = (j4 * BB + 16 * qq) * SKEW + 8 * tr
            for dd in range(8):
                val = plsc.load_gather(skew, [iota65 + (base + dd)])
                ostage[row, dd, pl.ds(16 * qq, 16)] = val

        for tr in range(8):
            pltpu.sync_copy(
                ostage.at[pl.ds(tr * GPC, GPC)],
                out_hbm.at[pl.ds(s * (8 * TCB) + tr * TCB + tq * GPC, GPC)],
            )
        return carry

    lax.fori_loop(0, PER_W, chunk_body, 0)


def kernel(x, table):
    xt = x.T.astype(jnp.int32)
    twide = jnp.pad(table, ((0, 0), (0, DIM)))
    t2 = twide.reshape(2 * 1000000, DIM)
    out3 = _build()(xt, t2)
    out5 = out3.reshape(SEQ, 8, TCB, 8, BB)
    return out5.transpose(2, 4, 0, 1, 3).reshape(BATCH, SEQ, DIM)


# speedup vs baseline: 2.8364x; 1.0162x over previous
import functools
import jax
import jax.numpy as jnp
from jax import lax
from jax.experimental import pallas as pl
from jax.experimental.pallas import tpu as pltpu
from jax.experimental.pallas import tpu_sc as plsc

DIM = 64
BATCH = 16384
SEQ = 50
SKEW = 65
NW = 32
BB = 128
GPC = 4
CB = BB * GPC
NCHUNK = SEQ * (BATCH // CB)
PER_W = NCHUNK // NW
TCB = BATCH // BB
TQ = BATCH // CB


@functools.cache
def _build():
    mesh = plsc.VectorSubcoreMesh(core_axis_name="c", subcore_axis_name="s")
    return pl.kernel(
        _emb_lookup,
        mesh=mesh,
        out_type=jax.ShapeDtypeStruct((SEQ * 8 * TCB, 8, BB), jnp.float32),
        scratch_types=[
            pltpu.VMEM((1, CB), jnp.int32),
            pltpu.VMEM((GPC, BB), jnp.int32),
            pltpu.VMEM((CB, DIM), jnp.float32),
            pltpu.VMEM((CB * SKEW,), jnp.float32),
            pltpu.VMEM((4 * 8, 8, BB), jnp.float32),
            pltpu.SemaphoreType.DMA,
        ],
        compiler_params=pltpu.CompilerParams(
            use_tc_tiling_on_sc=False, needs_layout_passes=False
        ),
    )


def _emb_lookup(xt_hbm, t2_hbm, out_hbm, idx_v, dbl_v, rows, skew, ostage, sem):
    cid = lax.axis_index("c")
    sid = lax.axis_index("s")
    wid = sid * 2 + cid
    c0 = wid * PER_W
    iota = lax.iota(jnp.int32, 16)
    iota65 = iota * SKEW

    def chunk_body(g, carry):
        c = c0 + g
        s = c // TQ
        tq = c % TQ

        pltpu.sync_copy(xt_hbm.at[pl.ds(s, 1), pl.ds(tq * CB, CB)], idx_v)
        for j in range(GPC):
            for k in range(BB // 16):
                dbl_v[j, pl.ds(16 * k, 16)] = (
                    idx_v[0, pl.ds(j * BB + 16 * k, 16)] * 2
                )
        copies = [
            pltpu.async_copy(
                t2_hbm.at[dbl_v.at[j]], rows.at[pl.ds(j * BB, BB)], sem
            )
            for j in range(GPC)
        ]

        for jq in range(GPC):
            copies[jq].wait()

            @plsc.parallel_loop(jq * BB, (jq + 1) * BB, unroll=4)
            def _pass1(bb):
                base = bb * SKEW
                for j in range(DIM // 16):
                    val = rows[bb, pl.ds(16 * j, 16)] * 8.0
                    plsc.store_scatter(skew, [base + 16 * j + iota], val)

        @plsc.parallel_loop(0, 8 * GPC * 8, unroll=2)
        def _pass2(o):
            row = o // 8
            qq = o % 8
            tr = row // GPC
            j4 = row % GPC
            base = (j4 * BB + 16 * qq) * SKEW + 8 * tr
            for dd in range(8):
                val = plsc.load_gather(skew, [iota65 + (base + dd)])
                ostage[row, dd, pl.ds(16 * qq, 16)] = val

        for tr in range(8):
            pltpu.sync_copy(
                ostage.at[pl.ds(tr * GPC, GPC)],
                out_hbm.at[pl.ds(s * (8 * TCB) + tr * TCB + tq * GPC, GPC)],
            )
        return carry

    lax.fori_loop(0, PER_W, chunk_body, 0)


def kernel(x, table):
    xt = x.T.astype(jnp.int32)
    twide = jnp.pad(table, ((0, 0), (0, DIM)))
    t2 = twide.reshape(2 * 1000000, DIM)
    out3 = _build()(xt, t2)
    out5 = out3.reshape(SEQ, 8, TCB, 8, BB)
    return out5.transpose(2, 4, 0, 1, 3).reshape(BATCH, SEQ, DIM)


# speedup vs baseline: 2.9338x; 1.0343x over previous
import functools
import jax
import jax.numpy as jnp
from jax import lax
from jax.experimental import pallas as pl
from jax.experimental.pallas import tpu as pltpu
from jax.experimental.pallas import tpu_sc as plsc

DIM = 64
BATCH = 16384
SEQ = 50
SKEW = 65
NW = 32
BB = 128
GPC = 4
CB = BB * GPC
NCHUNK = SEQ * (BATCH // CB)
PER_W = NCHUNK // NW
TCB = BATCH // BB
TQ = BATCH // CB


@functools.cache
def _build():
    mesh = plsc.VectorSubcoreMesh(core_axis_name="c", subcore_axis_name="s")
    return pl.kernel(
        _emb_lookup,
        mesh=mesh,
        out_type=jax.ShapeDtypeStruct((SEQ * 8 * TCB, 8, BB), jnp.float32),
        scratch_types=[
            pltpu.VMEM((1, CB), jnp.int32),
            pltpu.VMEM((GPC, BB), jnp.int32),
            pltpu.VMEM((CB, DIM), jnp.float32),
            pltpu.VMEM((CB * SKEW,), jnp.float32),
            pltpu.VMEM((4 * 8, 8, BB), jnp.float32),
            pltpu.SemaphoreType.DMA,
        ],
        compiler_params=pltpu.CompilerParams(
            use_tc_tiling_on_sc=False, needs_layout_passes=False
        ),
    )


def _emb_lookup(xt_hbm, t2_hbm, out_hbm, idx_v, dbl_v, rows, skew, ostage, sem):
    cid = lax.axis_index("c")
    sid = lax.axis_index("s")
    wid = sid * 2 + cid
    c0 = wid * PER_W
    iota = lax.iota(jnp.int32, 16)
    iota65 = iota * SKEW

    def stage(c):
        s = c // TQ
        tq = c % TQ
        pltpu.sync_copy(xt_hbm.at[pl.ds(s, 1), pl.ds(tq * CB, CB)], idx_v)
        for j in range(GPC):
            for k in range(BB // 16):
                dbl_v[j, pl.ds(16 * k, 16)] = (
                    idx_v[0, pl.ds(j * BB + 16 * k, 16)] * 2
                )
        for j in range(GPC):
            pltpu.async_copy(
                t2_hbm.at[dbl_v.at[j]], rows.at[pl.ds(j * BB, BB)], sem
            )

    stage(c0)

    def chunk_body(g, carry):
        c = c0 + g
        s = c // TQ
        tq = c % TQ

        for j in range(GPC):
            pltpu.make_async_copy(
                t2_hbm.at[dbl_v.at[j]], rows.at[pl.ds(j * BB, BB)], sem
            ).wait()

        @plsc.parallel_loop(0, CB, unroll=4)
        def _pass1(bb):
            base = bb * SKEW
            for j in range(DIM // 16):
                val = rows[bb, pl.ds(16 * j, 16)] * 8.0
                plsc.store_scatter(skew, [base + 16 * j + iota], val)

        @pl.when(g < PER_W - 1)
        def _prefetch():
            stage(c + 1)

        @plsc.parallel_loop(0, 8 * GPC * 8, unroll=2)
        def _pass2(o):
            row = o // 8
            qq = o % 8
            tr = row // GPC
            j4 = row % GPC
            base = (j4 * BB + 16 * qq) * SKEW + 8 * tr
            for dd in range(8):
                val = plsc.load_gather(skew, [iota65 + (base + dd)])
                ostage[row, dd, pl.ds(16 * qq, 16)] = val

        for tr in range(8):
            pltpu.sync_copy(
                ostage.at[pl.ds(tr * GPC, GPC)],
                out_hbm.at[pl.ds(s * (8 * TCB) + tr * TCB + tq * GPC, GPC)],
            )
        return carry

    lax.fori_loop(0, PER_W, chunk_body, 0)


def kernel(x, table):
    xt = x.T.astype(jnp.int32)
    twide = jnp.pad(table, ((0, 0), (0, DIM)))
    t2 = twide.reshape(2 * 1000000, DIM)
    out3 = _build()(xt, t2)
    out5 = out3.reshape(SEQ, 8, TCB, 8, BB)
    return out5.transpose(2, 4, 0, 1, 3).reshape(BATCH, SEQ, DIM)


# speedup vs baseline: 3.0977x; 1.0559x over previous
import functools
import jax
import jax.numpy as jnp
from jax import lax
from jax.experimental import pallas as pl
from jax.experimental.pallas import tpu as pltpu
from jax.experimental.pallas import tpu_sc as plsc

DIM = 64
BATCH = 16384
SEQ = 50
SKEW = 65
NW = 32
BB = 128
GPC = 4
CB = BB * GPC
NCHUNK = SEQ * (BATCH // CB)
PER_W = NCHUNK // NW
TCB = BATCH // BB
TQ = BATCH // CB


@functools.cache
def _build():
    mesh = plsc.VectorSubcoreMesh(core_axis_name="c", subcore_axis_name="s")
    return pl.kernel(
        _emb_lookup,
        mesh=mesh,
        out_type=jax.ShapeDtypeStruct((SEQ * 8 * TCB, 8, BB), jnp.float32),
        scratch_types=[
            pltpu.VMEM((1, CB), jnp.int32),
            pltpu.VMEM((GPC, BB), jnp.int32),
            pltpu.VMEM((CB, DIM), jnp.float32),
            pltpu.VMEM((CB * SKEW,), jnp.float32),
            pltpu.VMEM((4 * 8, 8, BB), jnp.float32),
            pltpu.SemaphoreType.DMA,
            pltpu.SemaphoreType.DMA,
        ],
        compiler_params=pltpu.CompilerParams(
            use_tc_tiling_on_sc=False, needs_layout_passes=False
        ),
    )


def _emb_lookup(xt_hbm, t2_hbm, out_hbm, idx_v, dbl_v, rows, skew, ostage, sem, sem2):
    cid = lax.axis_index("c")
    sid = lax.axis_index("s")
    wid = sid * 2 + cid
    c0 = wid * PER_W
    iota = lax.iota(jnp.int32, 16)
    iota65 = iota * SKEW

    def stage(c):
        s = c // TQ
        tq = c % TQ
        pltpu.sync_copy(xt_hbm.at[pl.ds(s, 1), pl.ds(tq * CB, CB)], idx_v)
        for j in range(GPC):
            for k in range(BB // 16):
                dbl_v[j, pl.ds(16 * k, 16)] = (
                    idx_v[0, pl.ds(j * BB + 16 * k, 16)] * 2
                )
        for j in range(GPC):
            pltpu.async_copy(
                t2_hbm.at[dbl_v.at[j]], rows.at[pl.ds(j * BB, BB)], sem
            )

    stage(c0)

    def chunk_body(g, carry):
        c = c0 + g
        s = c // TQ
        tq = c % TQ

        for j in range(GPC):
            pltpu.make_async_copy(
                t2_hbm.at[dbl_v.at[j]], rows.at[pl.ds(j * BB, BB)], sem
            ).wait()

        @plsc.parallel_loop(0, CB, unroll=4)
        def _pass1(bb):
            base = bb * SKEW
            for j in range(DIM // 16):
                val = rows[bb, pl.ds(16 * j, 16)] * 8.0
                plsc.store_scatter(skew, [base + 16 * j + iota], val)

        @pl.when(g < PER_W - 1)
        def _prefetch():
            stage(c + 1)

        @pl.when(g > 0)
        def _drain_out():
            for tr in range(8):
                pltpu.make_async_copy(
                    ostage.at[pl.ds(tr * GPC, GPC)],
                    out_hbm.at[pl.ds(s * (8 * TCB) + tr * TCB + tq * GPC, GPC)],
                    sem2,
                ).wait()

        @plsc.parallel_loop(0, 8 * GPC * 8, unroll=2)
        def _pass2(o):
            row = o // 8
            qq = o % 8
            tr = row // GPC
            j4 = row % GPC
            base = (j4 * BB + 16 * qq) * SKEW + 8 * tr
            for dd in range(8):
                val = plsc.load_gather(skew, [iota65 + (base + dd)])
                ostage[row, dd, pl.ds(16 * qq, 16)] = val

        for tr in range(8):
            pltpu.async_copy(
                ostage.at[pl.ds(tr * GPC, GPC)],
                out_hbm.at[pl.ds(s * (8 * TCB) + tr * TCB + tq * GPC, GPC)],
                sem2,
            )
        return carry

    lax.fori_loop(0, PER_W, chunk_body, 0)

    cl = c0 + PER_W - 1
    sl = cl // TQ
    tql = cl % TQ
    for tr in range(8):
        pltpu.make_async_copy(
            ostage.at[pl.ds(tr * GPC, GPC)],
            out_hbm.at[pl.ds(sl * (8 * TCB) + tr * TCB + tql * GPC, GPC)],
            sem2,
        ).wait()


def kernel(x, table):
    xt = x.T.astype(jnp.int32)
    twide = jnp.pad(table, ((0, 0), (0, DIM)))
    t2 = twide.reshape(2 * 1000000, DIM)
    out3 = _build()(xt, t2)
    out5 = out3.reshape(SEQ, 8, TCB, 8, BB)
    return out5.transpose(2, 4, 0, 1, 3).reshape(BATCH, SEQ, DIM)


# speedup vs baseline: 3.2457x; 1.0478x over previous
import functools
import jax
import jax.numpy as jnp
from jax import lax
from jax.experimental import pallas as pl
from jax.experimental.pallas import tpu as pltpu
from jax.experimental.pallas import tpu_sc as plsc

DIM = 64
BATCH = 16384
SEQ = 50
SKEW = 65
NW = 32
BB = 128
GPC = 4
CB = BB * GPC
NCHUNK = SEQ * (BATCH // CB)
PER_W = NCHUNK // NW
TCB = BATCH // BB
TQ = BATCH // CB


@functools.cache
def _build():
    mesh = plsc.VectorSubcoreMesh(core_axis_name="c", subcore_axis_name="s")
    return pl.kernel(
        _emb_lookup,
        mesh=mesh,
        out_type=jax.ShapeDtypeStruct((SEQ * 8 * TCB, 8, BB), jnp.float32),
        scratch_types=[
            pltpu.VMEM((1, CB), jnp.int32),
            pltpu.VMEM((GPC, BB), jnp.int32),
            pltpu.VMEM((CB, DIM), jnp.float32),
            pltpu.VMEM((CB * SKEW,), jnp.float32),
            pltpu.VMEM((4 * 8, 8, BB), jnp.float32),
            pltpu.SemaphoreType.DMA,
            pltpu.SemaphoreType.DMA,
        ],
        compiler_params=pltpu.CompilerParams(
            use_tc_tiling_on_sc=False, needs_layout_passes=False
        ),
    )


def _emb_lookup(xt_hbm, t2_hbm, out_hbm, idx_v, dbl_v, rows, skew, ostage, sem, sem2):
    cid = lax.axis_index("c")
    sid = lax.axis_index("s")
    wid = sid * 2 + cid
    c0 = wid * PER_W
    iota = lax.iota(jnp.int32, 16)
    iota65 = iota * SKEW

    def stage(c):
        s = c // TQ
        tq = c % TQ
        pltpu.sync_copy(xt_hbm.at[pl.ds(s, 1), pl.ds(tq * CB, CB)], idx_v)
        for j in range(GPC):
            for k in range(BB // 16):
                dbl_v[j, pl.ds(16 * k, 16)] = (
                    idx_v[0, pl.ds(j * BB + 16 * k, 16)] * 2
                )
        for j in range(GPC):
            pltpu.async_copy(
                t2_hbm.at[dbl_v.at[j]], rows.at[pl.ds(j * BB, BB)], sem
            )

    stage(c0)

    def chunk_body(g, carry):
        c = c0 + g
        s = c // TQ
        tq = c % TQ

        for j in range(GPC):
            pltpu.make_async_copy(
                t2_hbm.at[dbl_v.at[j]], rows.at[pl.ds(j * BB, BB)], sem
            ).wait()

        @plsc.parallel_loop(0, CB, unroll=8)
        def _pass1(bb):
            base = bb * SKEW
            for j in range(DIM // 16):
                val = rows[bb, pl.ds(16 * j, 16)] * 8.0
                plsc.store_scatter(skew, [base + 16 * j + iota], val)

        @pl.when(g < PER_W - 1)
        def _prefetch():
            stage(c + 1)

        @pl.when(g > 0)
        def _drain_out():
            for tr in range(8):
                pltpu.make_async_copy(
                    ostage.at[pl.ds(tr * GPC, GPC)],
                    out_hbm.at[pl.ds(s * (8 * TCB) + tr * TCB + tq * GPC, GPC)],
                    sem2,
                ).wait()

        @plsc.parallel_loop(0, 8 * GPC * 8, unroll=4)
        def _pass2(o):
            row = o // 8
            qq = o % 8
            tr = row // GPC
            j4 = row % GPC
            base = (j4 * BB + 16 * qq) * SKEW + 8 * tr
            for dd in range(8):
                val = plsc.load_gather(skew, [iota65 + (base + dd)])
                ostage[row, dd, pl.ds(16 * qq, 16)] = val

        for tr in range(8):
            pltpu.async_copy(
                ostage.at[pl.ds(tr * GPC, GPC)],
                out_hbm.at[pl.ds(s * (8 * TCB) + tr * TCB + tq * GPC, GPC)],
                sem2,
            )
        return carry

    lax.fori_loop(0, PER_W, chunk_body, 0)

    cl = c0 + PER_W - 1
    sl = cl // TQ
    tql = cl % TQ
    for tr in range(8):
        pltpu.make_async_copy(
            ostage.at[pl.ds(tr * GPC, GPC)],
            out_hbm.at[pl.ds(sl * (8 * TCB) + tr * TCB + tql * GPC, GPC)],
            sem2,
        ).wait()


def kernel(x, table):
    xt = x.T.astype(jnp.int32)
    twide = jnp.pad(table, ((0, 0), (0, DIM)))
    t2 = twide.reshape(2 * 1000000, DIM)
    out3 = _build()(xt, t2)
    out5 = out3.reshape(SEQ, 8, TCB, 8, BB)
    return out5.transpose(2, 4, 0, 1, 3).reshape(BATCH, SEQ, DIM)


# speedup vs baseline: 3.2930x; 1.0146x over previous
import functools
import jax
import jax.numpy as jnp
from jax import lax
from jax.experimental import pallas as pl
from jax.experimental.pallas import tpu as pltpu
from jax.experimental.pallas import tpu_sc as plsc

DIM = 64
BATCH = 16384
SEQ = 50
SKEW = 65
NW = 32
BB = 128
GPC = 4
CB = BB * GPC
NCHUNK = SEQ * (BATCH // CB)
PER_W = NCHUNK // NW
TCB = BATCH // BB
TQ = BATCH // CB


@functools.cache
def _build():
    mesh = plsc.VectorSubcoreMesh(core_axis_name="c", subcore_axis_name="s")
    return pl.kernel(
        _emb_lookup,
        mesh=mesh,
        out_type=jax.ShapeDtypeStruct((SEQ * 8 * TCB, 8, BB), jnp.float32),
        scratch_types=[
            pltpu.VMEM((1, CB), jnp.int32),
            pltpu.VMEM((GPC, BB), jnp.int32),
            pltpu.VMEM((CB, DIM), jnp.float32),
            pltpu.VMEM((CB * SKEW,), jnp.float32),
            pltpu.VMEM((4 * 8, 8, BB), jnp.float32),
            pltpu.SemaphoreType.DMA,
            pltpu.SemaphoreType.DMA,
        ],
        compiler_params=pltpu.CompilerParams(
            use_tc_tiling_on_sc=False, needs_layout_passes=False
        ),
    )


def _emb_lookup(xt_hbm, t2_hbm, out_hbm, idx_v, dbl_v, rows, skew, ostage, sem, sem2):
    cid = lax.axis_index("c")
    sid = lax.axis_index("s")
    wid = sid * 2 + cid
    c0 = wid * PER_W
    iota = lax.iota(jnp.int32, 16)
    iota65 = iota * SKEW

    def stage(c):
        s = c // TQ
        tq = c % TQ
        pltpu.sync_copy(xt_hbm.at[pl.ds(s, 1), pl.ds(tq * CB, CB)], idx_v)
        for j in range(GPC):
            for k in range(BB // 16):
                dbl_v[j, pl.ds(16 * k, 16)] = (
                    idx_v[0, pl.ds(j * BB + 16 * k, 16)] * 2
                )
        for j in range(GPC):
            pltpu.async_copy(
                t2_hbm.at[dbl_v.at[j]], rows.at[pl.ds(j * BB, BB)], sem
            )

    stage(c0)

    def chunk_body(g, carry):
        c = c0 + g
        s = c // TQ
        tq = c % TQ

        for j in range(GPC):
            pltpu.make_async_copy(
                t2_hbm.at[dbl_v.at[j]], rows.at[pl.ds(j * BB, BB)], sem
            ).wait()

        @plsc.parallel_loop(0, CB, unroll=16)
        def _pass1(bb):
            base = bb * SKEW
            for j in range(DIM // 16):
                val = rows[bb, pl.ds(16 * j, 16)] * 8.0
                plsc.store_scatter(skew, [base + 16 * j + iota], val)

        @pl.when(g < PER_W - 1)
        def _prefetch():
            stage(c + 1)

        @pl.when(g > 0)
        def _drain_out():
            for tr in range(8):
                pltpu.make_async_copy(
                    ostage.at[pl.ds(tr * GPC, GPC)],
                    out_hbm.at[pl.ds(s * (8 * TCB) + tr * TCB + tq * GPC, GPC)],
                    sem2,
                ).wait()

        @plsc.parallel_loop(0, 8 * GPC * 8, unroll=8)
        def _pass2(o):
            row = o // 8
            qq = o % 8
            tr = row // GPC
            j4 = row % GPC
            base = (j4 * BB + 16 * qq) * SKEW + 8 * tr
            for dd in range(8):
                val = plsc.load_gather(skew, [iota65 + (base + dd)])
                ostage[row, dd, pl.ds(16 * qq, 16)] = val

        for tr in range(8):
            pltpu.async_copy(
                ostage.at[pl.ds(tr * GPC, GPC)],
                out_hbm.at[pl.ds(s * (8 * TCB) + tr * TCB + tq * GPC, GPC)],
                sem2,
            )
        return carry

    lax.fori_loop(0, PER_W, chunk_body, 0)

    cl = c0 + PER_W - 1
    sl = cl // TQ
    tql = cl % TQ
    for tr in range(8):
        pltpu.make_async_copy(
            ostage.at[pl.ds(tr * GPC, GPC)],
            out_hbm.at[pl.ds(sl * (8 * TCB) + tr * TCB + tql * GPC, GPC)],
            sem2,
        ).wait()


def kernel(x, table):
    xt = x.T.astype(jnp.int32)
    twide = jnp.pad(table, ((0, 0), (0, DIM)))
    t2 = twide.reshape(2 * 1000000, DIM)
    out3 = _build()(xt, t2)
    out5 = out3.reshape(SEQ, 8, TCB, 8, BB)
    return out5.transpose(2, 4, 0, 1, 3).reshape(BATCH, SEQ, DIM)
